# Initial kernel scaffold; baseline (speedup 1.0000x reference)
import jax
import jax.numpy as jnp
from jax.experimental import pallas as pl


def kernel(X, embedding, offset):
    raise NotImplementedError("write your pallas kernel here")


# speedup vs baseline: 1.7129x; 1.7129x over previous
import jax
import jax.numpy as jnp
from jax.experimental import pallas as pl
from jax.experimental.pallas import tpu as pltpu


def _body(off_ref, x_ref, emb_hbm, o_ref, e_scr, sems):
    l, b = pl.program_id(0), pl.program_id(1)
    nl = pl.num_programs(0)
    BL = e_scr.shape[1]

    def _copy(lblk, slot):
        return pltpu.make_async_copy(
            emb_hbm.at[pl.ds(pl.multiple_of(off_ref[0] + lblk * BL, 8), BL)],
            e_scr.at[slot],
            sems.at[slot],
        )

    @pl.when(jnp.logical_and(l == 0, b == 0))
    def _():
        _copy(0, 0).start()

    @pl.when(b == 0)
    def _():
        @pl.when(l + 1 < nl)
        def _():
            _copy(l + 1, (l + 1) % 2).start()
        _copy(l, l % 2).wait()

    o_ref[...] = x_ref[...] + e_scr[l % 2]


def kernel(X, embedding, offset):
    B, L, D = X.shape
    BL = 512
    off = jnp.asarray(offset, jnp.int32).reshape(1)
    grid = (L // BL, B)
    out = pl.pallas_call(
        _body,
        grid_spec=pltpu.PrefetchScalarGridSpec(
            num_scalar_prefetch=1,
            grid=grid,
            in_specs=[
                pl.BlockSpec((1, BL, D), lambda l, b, off_r: (b, l, 0)),
                pl.BlockSpec(memory_space=pl.ANY),
            ],
            out_specs=pl.BlockSpec((1, BL, D), lambda l, b, off_r: (b, l, 0)),
            scratch_shapes=[
                pltpu.VMEM((2, BL, D), jnp.float32),
                pltpu.SemaphoreType.DMA((2,)),
            ],
        ),
        out_shape=jax.ShapeDtypeStruct(X.shape, X.dtype),
        compiler_params=pltpu.CompilerParams(
            dimension_semantics=("arbitrary", "arbitrary"),
        ),
    )(off, X, embedding)
    return out
